# baseline (speedup 1.0000x reference)
import functools

import jax
import jax.numpy as jnp
from jax import lax
from jax.experimental import pallas as pl
from jax.experimental.pallas import tpu as pltpu
from jax.experimental.pallas import tpu_sc as plsc

GRID = 512
C = 48
CP = 64
NPLANES = 3
NROWS = NPLANES * GRID * GRID
NCORES = 2
NSUB = 16
NW = NCORES * NSUB
K = 128
XSUP = 8
PLANE_AXES = ((1, 2), (0, 1), (2, 0))


def _worker_body(n_points, x_hbm, table_hbm, out_hbm,
                 xv, idxv0, wref0, rows0, outv0, idxv1, wref1, rows1, outv1,
                 gsem0, gsem1, osem0, osem1):
    wid = lax.axis_index("c") * NSUB + lax.axis_index("s")
    pw = n_points // NW
    nchunks = pw // K
    wbase = wid * pw
    bufs = ((idxv0, wref0, rows0, outv0, gsem0, osem0),
            (idxv1, wref1, rows1, outv1, gsem1, osem1))

    lanes = lax.iota(jnp.int32, 16)
    col_even0 = lanes * 2
    col_odd0 = lanes * 2 + 1
    col_even1 = lanes * 2 + 32
    col_odd1 = lanes * 2 + 33
    lo_mask = lanes < 8

    def stage_x(sup_base):
        for p in range(3):
            pltpu.sync_copy(
                x_hbm.at[pl.ds(p * n_points + sup_base, XSUP * K)], xv.at[p])

    def compute_and_fire(t, idxv, wref, rows, gsem):
        col0 = (t % XSUP) * K
        for g in range(K // 16):
            slx = pl.ds(col0 + g * 16, 16)
            sl = pl.ds(g * 16, 16)
            xc = [xv[p, slx] for p in range(3)]
            for p, (a, b) in enumerate(PLANE_AXES):
                u = xc[a] * 128.0 + 256.0
                v = xc[b] * 128.0 + 256.0
                iu = u.astype(jnp.int32)
                iv = v.astype(jnp.int32)
                wu1 = u - iu.astype(jnp.float32)
                wv1 = v - iv.astype(jnp.float32)
                wu0 = 1.0 - wu1
                wv0 = (1.0 - wv1) * (1.0 / 3.0)
                wv1 = wv1 * (1.0 / 3.0)
                u00 = iv * GRID + iu + p * (GRID * GRID)
                idxv[2 * p + 0, sl] = u00
                idxv[2 * p + 1, sl] = u00 + GRID
                wref[4 * p + 0, sl] = wu0 * wv0
                wref[4 * p + 1, sl] = wu1 * wv0
                wref[4 * p + 2, sl] = wu0 * wv1
                wref[4 * p + 3, sl] = wu1 * wv1
        for r in range(6):
            pltpu.async_copy(table_hbm.at[idxv.at[r]], rows.at[r], gsem)

    def wait_gathers(idxv, rows, gsem):
        for r in range(6):
            pltpu.make_async_copy(table_hbm.at[idxv.at[r]], rows.at[r], gsem).wait()

    def accumulate(rows, wref, outv):
        def acc_body(g2, _):
            sl = pl.ds(g2 * 16, 16)
            wv = [wref[r, sl] for r in range(12)]
            for j in range(16):
                i = g2 * 16 + j
                acc = [jnp.zeros((16,), jnp.float32) for _ in range(4)]
                for p in range(3):
                    for dy in range(2):
                        for dx in range(2):
                            w = wv[4 * p + 2 * dy + dx][j]
                            off = dx * CP
                            vA = rows[2 * p + dy, i, pl.ds(off, 32)]
                            vB = rows[2 * p + dy, i, pl.ds(off + 32, 32)]
                            e0, o0 = plsc.unpack(
                                vA, format=plsc.PackFormat.INTERLEAVED)
                            e1, o1 = plsc.unpack(
                                vB, format=plsc.PackFormat.INTERLEAVED)
                            acc[0] = acc[0] + w * e0
                            acc[1] = acc[1] + w * o0
                            acc[2] = acc[2] + w * e1
                            acc[3] = acc[3] + w * o1
                base = jnp.full((16,), i * C, jnp.int32)
                plsc.store_scatter(outv, [base + col_even0], acc[0])
                plsc.store_scatter(outv, [base + col_odd0], acc[1])
                plsc.store_scatter(outv, [base + col_even1], acc[2], mask=lo_mask)
                plsc.store_scatter(outv, [base + col_odd1], acc[3], mask=lo_mask)
            return 0

        lax.fori_loop(0, K // 16, acc_body, 0)

    stage_x(wbase)
    compute_and_fire(0, bufs[0][0], bufs[0][1], bufs[0][2], bufs[0][4])

    def chunk_body(t2, _):
        for phase in range(2):
            t = t2 * 2 + phase
            idxv, wref, rows, outv, gsem, osem = bufs[phase]
            nidxv, nwref, nrows, _, ngsem, _ = bufs[1 - phase]
            tn = t + 1

            @pl.when(tn < nchunks)
            def _():
                @pl.when(tn % XSUP == 0)
                def _():
                    stage_x(wbase + tn * K)
                compute_and_fire(tn, nidxv, nwref, nrows, ngsem)

            wait_gathers(idxv, rows, gsem)

            @pl.when(t >= 2)
            def _():
                pltpu.make_async_copy(outv, out_hbm.at[pl.ds(0, K * C)], osem).wait()

            accumulate(rows, wref, outv)
            pltpu.async_copy(
                outv, out_hbm.at[pl.ds((wbase + t * K) * C, K * C)], osem)
        return 0

    lax.fori_loop(0, nchunks // 2, chunk_body, 0)

    for phase in range(2):
        _, _, _, outv, _, osem = bufs[phase]
        pltpu.make_async_copy(outv, out_hbm.at[pl.ds(0, K * C)], osem).wait()


@functools.partial(jax.jit, static_argnames=("n_points",))
def _triplane_sc(xt, table, n_points):
    mesh = plsc.VectorSubcoreMesh(
        core_axis_name="c", subcore_axis_name="s",
        num_cores=NCORES, num_subcores=NSUB,
    )
    body = functools.partial(_worker_body, n_points)
    buf = [
        pltpu.VMEM((6, K), jnp.int32),
        pltpu.VMEM((12, K), jnp.float32),
        pltpu.VMEM((6, K, 2 * CP), jnp.bfloat16),
        pltpu.VMEM((K * C,), jnp.float32),
    ]
    return pl.kernel(
        body,
        out_type=jax.ShapeDtypeStruct((n_points * C,), jnp.float32),
        mesh=mesh,
        scratch_types=(
            [pltpu.VMEM((3, XSUP * K), jnp.float32)]
            + buf + buf
            + [pltpu.SemaphoreType.DMA] * 4
        ),
        compiler_params=pltpu.CompilerParams(
            use_tc_tiling_on_sc=False, needs_layout_passes=False),
    )(xt, table)


def kernel(x, features_2d):
    n = x.shape[0]
    xt = lax.reduce_precision(x.T, exponent_bits=8, mantissa_bits=7)
    tab = features_2d.reshape(NROWS, C).astype(jnp.bfloat16)
    tab = jnp.pad(tab, ((0, 0), (0, CP - C)))
    tab_next = jnp.concatenate(
        [tab[1:], jnp.zeros((1, CP), jnp.bfloat16)], axis=0)
    t_over = jnp.concatenate([tab, tab_next], axis=1)
    out = _triplane_sc(xt.reshape(-1), t_over, n)
    return out.reshape(n, C)


# speedup vs baseline: 1.3331x; 1.3331x over previous
import functools

import jax
import jax.numpy as jnp
from jax import lax
from jax.experimental import pallas as pl
from jax.experimental.pallas import tpu as pltpu
from jax.experimental.pallas import tpu_sc as plsc

GRID = 512
C = 48
NPLANES = 3
NROWS = NPLANES * GRID * GRID
NCORES = 2
NSUB = 16
NW = NCORES * NSUB
K = 128
XSUP = 8
PLANE_AXES = ((1, 2), (0, 1), (2, 0))


def _worker_body(n_points, x_hbm, table_hbm, out_hbm,
                 xv, idxv0, wref0, rows0, outv0, idxv1, wref1, rows1, outv1,
                 gsem0, gsem1, osem0, osem1):
    wid = lax.axis_index("c") * NSUB + lax.axis_index("s")
    pw = n_points // NW
    nchunks = pw // K
    wbase = wid * pw
    bufs = ((idxv0, wref0, rows0, outv0, gsem0, osem0),
            (idxv1, wref1, rows1, outv1, gsem1, osem1))

    lanes = lax.iota(jnp.int32, 16)
    col_even0 = lanes * 2
    col_odd0 = lanes * 2 + 1
    col_even1 = lanes * 2 + 16
    col_odd1 = lanes * 2 + 17
    hi_mask = lanes >= 8

    def stage_x(sup_base):
        for p in range(3):
            pltpu.sync_copy(
                x_hbm.at[pl.ds(p * n_points + sup_base, XSUP * K)], xv.at[p])

    def compute_and_fire(t, idxv, wref, rows, gsem):
        col0 = (t % XSUP) * K
        for g in range(K // 16):
            slx = pl.ds(col0 + g * 16, 16)
            sl = pl.ds(g * 16, 16)
            xc = [xv[p, slx] for p in range(3)]
            for p, (a, b) in enumerate(PLANE_AXES):
                u = xc[a] * 128.0 + 256.0
                v = xc[b] * 128.0 + 256.0
                iu = u.astype(jnp.int32)
                iv = v.astype(jnp.int32)
                wu1 = u - iu.astype(jnp.float32)
                wv1 = v - iv.astype(jnp.float32)
                wu0 = 1.0 - wu1
                wv0 = (1.0 - wv1) * (1.0 / 3.0)
                wv1 = wv1 * (1.0 / 3.0)
                r00 = iv * GRID + iu + p * (GRID * GRID)
                idxv[4 * p + 0, sl] = r00
                idxv[4 * p + 1, sl] = r00 + 1
                idxv[4 * p + 2, sl] = r00 + GRID
                idxv[4 * p + 3, sl] = r00 + GRID + 1
                wref[4 * p + 0, sl] = wu0 * wv0
                wref[4 * p + 1, sl] = wu1 * wv0
                wref[4 * p + 2, sl] = wu0 * wv1
                wref[4 * p + 3, sl] = wu1 * wv1
        for r in range(12):
            pltpu.async_copy(table_hbm.at[idxv.at[r]], rows.at[r], gsem)

    def wait_gathers(idxv, rows, gsem):
        for r in range(12):
            pltpu.make_async_copy(table_hbm.at[idxv.at[r]], rows.at[r], gsem).wait()

    def accumulate(rows, wref, outv):
        def acc_body(g2, _):
            sl = pl.ds(g2 * 16, 16)
            wv = [wref[r, sl] for r in range(12)]
            for j in range(16):
                i = g2 * 16 + j
                acc = [jnp.zeros((16,), jnp.float32) for _ in range(4)]
                for r in range(12):
                    w = wv[r][j]
                    vA = rows[r, i, pl.ds(0, 32)]
                    vB = rows[r, i, pl.ds(16, 32)]
                    e0, o0 = plsc.unpack(vA, format=plsc.PackFormat.INTERLEAVED)
                    e1, o1 = plsc.unpack(vB, format=plsc.PackFormat.INTERLEAVED)
                    acc[0] = acc[0] + w * e0
                    acc[1] = acc[1] + w * o0
                    acc[2] = acc[2] + w * e1
                    acc[3] = acc[3] + w * o1
                base = jnp.full((16,), i * C, jnp.int32)
                plsc.store_scatter(outv, [base + col_even0], acc[0])
                plsc.store_scatter(outv, [base + col_odd0], acc[1])
                plsc.store_scatter(outv, [base + col_even1], acc[2], mask=hi_mask)
                plsc.store_scatter(outv, [base + col_odd1], acc[3], mask=hi_mask)
            return 0

        lax.fori_loop(0, K // 16, acc_body, 0)

    stage_x(wbase)
    compute_and_fire(0, bufs[0][0], bufs[0][1], bufs[0][2], bufs[0][4])

    def chunk_body(t2, _):
        for phase in range(2):
            t = t2 * 2 + phase
            idxv, wref, rows, outv, gsem, osem = bufs[phase]
            nidxv, nwref, nrows, _, ngsem, _ = bufs[1 - phase]
            tn = t + 1

            @pl.when(tn < nchunks)
            def _():
                @pl.when(tn % XSUP == 0)
                def _():
                    stage_x(wbase + tn * K)
                compute_and_fire(tn, nidxv, nwref, nrows, ngsem)

            wait_gathers(idxv, rows, gsem)

            @pl.when(t >= 2)
            def _():
                pltpu.make_async_copy(outv, out_hbm.at[pl.ds(0, K * C)], osem).wait()

            accumulate(rows, wref, outv)
            pltpu.async_copy(
                outv, out_hbm.at[pl.ds((wbase + t * K) * C, K * C)], osem)
        return 0

    lax.fori_loop(0, nchunks // 2, chunk_body, 0)

    for phase in range(2):
        _, _, _, outv, _, osem = bufs[phase]
        pltpu.make_async_copy(outv, out_hbm.at[pl.ds(0, K * C)], osem).wait()


@functools.partial(jax.jit, static_argnames=("n_points",))
def _triplane_sc(xt, table, n_points):
    mesh = plsc.VectorSubcoreMesh(
        core_axis_name="c", subcore_axis_name="s",
        num_cores=NCORES, num_subcores=NSUB,
    )
    body = functools.partial(_worker_body, n_points)
    buf = [
        pltpu.VMEM((12, K), jnp.int32),
        pltpu.VMEM((12, K), jnp.float32),
        pltpu.VMEM((12, K, C), jnp.bfloat16),
        pltpu.VMEM((K * C,), jnp.float32),
    ]
    return pl.kernel(
        body,
        out_type=jax.ShapeDtypeStruct((n_points * C,), jnp.float32),
        mesh=mesh,
        scratch_types=(
            [pltpu.VMEM((3, XSUP * K), jnp.float32)]
            + buf + buf
            + [pltpu.SemaphoreType.DMA] * 4
        ),
        compiler_params=pltpu.CompilerParams(
            use_tc_tiling_on_sc=False, needs_layout_passes=False),
    )(xt, table)


def kernel(x, features_2d):
    n = x.shape[0]
    xt = lax.reduce_precision(x.T, exponent_bits=8, mantissa_bits=7)
    table = features_2d.reshape(NROWS, C).astype(jnp.bfloat16)
    out = _triplane_sc(xt.reshape(-1), table, n)
    one = 1.0 + 0.0 * x[0, 0]
    return out.reshape(n, C) * one
